# baseline (speedup 1.0000x reference)
import jax
import jax.numpy as jnp
from jax import lax
from jax.experimental import pallas as pl
from jax.experimental.pallas import tpu as pltpu
from jax.experimental.pallas import tpu_sc as plsc

B, T, D = 4, 4096, 2048
NC, NS = 2, 16
NW = NC * NS
TR = T // NW
CT = 8
NCH = TR // CT
CHUNK = CT * D
NSTEP = NCH * B


def _sc_body(x_hbm, t_hbm, o_hbm, xbuf, tbuf, sem0, sem1):
    sems = [sem0, sem1]
    wid = lax.axis_index("s") * NC + lax.axis_index("c")
    base = wid * (TR * D)

    def in_copy(s):
        b, c = s % B, s // B
        u = s % 2
        return pltpu.make_async_copy(
            x_hbm.at[b, pl.ds(base + c * CHUNK, CHUNK)], xbuf.at[u], sems[u])

    in_copy(0).start()
    for s in range(NSTEP):
        u, b, c = s % 2, s % B, s // B
        if b == 0:
            pltpu.sync_copy(t_hbm.at[pl.ds(base + c * CHUNK, CHUNK)], tbuf)
        if s + 1 < NSTEP:
            in_copy(s + 1).start()
        in_copy(s).wait()

        def add8(i, carry, _u=u):
            for k in range(8):
                off = (i * 8 + k) * 16
                plsc.addupdate(xbuf.at[_u, pl.ds(off, 16)],
                               tbuf[pl.ds(off, 16)])
            return carry

        lax.fori_loop(0, CHUNK // 128, add8, 0)
        pltpu.sync_copy(xbuf.at[u], o_hbm.at[b, pl.ds(base + c * CHUNK, CHUNK)])


def kernel(x, table):
    xf = x.reshape(B, T * D)
    tf = table.reshape(T * D)
    k = pl.kernel(
        _sc_body,
        mesh=plsc.VectorSubcoreMesh(core_axis_name="c", subcore_axis_name="s"),
        out_type=jax.ShapeDtypeStruct((B, T * D), jnp.float32),
        scratch_types=[
            pltpu.VMEM((2, CHUNK), jnp.float32),
            pltpu.VMEM((CHUNK,), jnp.float32),
            pltpu.SemaphoreType.DMA,
            pltpu.SemaphoreType.DMA,
        ],
    )
    return k(xf, tf).reshape(B, T, D)


# speedup vs baseline: 6.3456x; 6.3456x over previous
import jax
import jax.numpy as jnp
from jax.experimental import pallas as pl


def _add_body(x_ref, t_ref, o_ref):
    o_ref[...] = x_ref[...] + t_ref[...]


def kernel(x, table):
    B, T, D = x.shape
    BS = 1024
    grid = (T // BS, B)
    return pl.pallas_call(
        _add_body,
        grid=grid,
        in_specs=[
            pl.BlockSpec((1, BS, D), lambda s, b: (b, s, 0)),
            pl.BlockSpec((BS, D), lambda s, b: (s, 0)),
        ],
        out_specs=pl.BlockSpec((1, BS, D), lambda s, b: (b, s, 0)),
        out_shape=jax.ShapeDtypeStruct(x.shape, x.dtype),
    )(x, table)
